# baseline (speedup 1.0000x reference)
import functools

import jax
import jax.numpy as jnp
from jax import lax
from jax.experimental import pallas as pl
from jax.experimental.pallas import tpu as pltpu
from jax.experimental.pallas import tpu_sc as plsc

_VOCAB = 100000
_D = 128
_NEG = 50
_NEG_PAD = 64
_B = 16384

_NW = 32
_BPW = _B // _NW
_CB = 16
_NCHUNK = _BPW // _CB
_ROWS = _CB * _NEG
_GCH = [(j * 128, 128) for j in range(6)] + [(768, 32)]


_GATHER_DNUMS = lax.GatherDimensionNumbers(
    offset_dims=(), collapsed_slice_dims=(0,), start_index_map=(0,))


def _lane_bcast(vec, lane_idx):
    idx = jnp.full((16, 1), lane_idx, jnp.int32)
    return lax.gather(vec, idx, _GATHER_DNUMS, slice_sizes=(1,),
                      mode=lax.GatherScatterMode.PROMISE_IN_BOUNDS)


def _body(t_ref, c_ref, ctab, ttab, out_ref,
          t_idx, c_idx, ctx_v, tgt_v, z_v, sem):
    wid = lax.axis_index("s") * 2 + lax.axis_index("c")
    lane = lax.iota(jnp.int32, 16)

    def chunk_body(ch, _):
        base = wid * _BPW + ch * _CB

        pltpu.sync_copy(t_ref.at[pl.ds(base, _CB)], t_idx)
        pltpu.sync_copy(c_ref.at[pl.ds(base * _NEG, _ROWS)], c_idx)

        copies = [pltpu.async_copy(ctab.at[t_idx], ctx_v, sem)]
        for off, sz in _GCH:
            copies.append(pltpu.async_copy(
                ttab.at[c_idx.at[pl.ds(off, sz)]],
                tgt_v.at[pl.ds(off, sz)], sem))
        for cp in copies:
            cp.wait()

        def b_body(b, _):
            ngrp = _NEG_PAD // 16
            rows = [b * _NEG + jnp.where(g * 16 + lane < _NEG,
                                         g * 16 + lane, 0)
                    for g in range(ngrp)]
            accs = [jnp.zeros((16,), jnp.float32) for _ in range(ngrp)]
            for k in range(_D // 16):
                ctx_k = ctx_v[b, pl.ds(k * 16, 16)]
                for dd in range(16):
                    d = k * 16 + dd
                    bc = _lane_bcast(ctx_k, dd)
                    didx = jnp.full((16,), d, jnp.int32)
                    for g in range(ngrp):
                        col = plsc.load_gather(tgt_v, [rows[g], didx])
                        accs[g] = accs[g] + bc * col
            zs = [jnp.where(g * 16 + lane < _NEG, accs[g],
                            jnp.float32(-jnp.inf)) for g in range(ngrp)]
            m = jnp.max(jnp.maximum(jnp.maximum(zs[0], zs[1]),
                                    jnp.maximum(zs[2], zs[3])))
            es = [jnp.exp(zj - m) for zj in zs]
            s = jnp.sum(es[0] + es[1] + es[2] + es[3])
            for j in range(ngrp):
                z_v[b, pl.ds(j * 16, 16)] = es[j] / s
            return 0

        lax.fori_loop(0, _CB, b_body, 0)
        pltpu.sync_copy(z_v, out_ref.at[pl.ds(base, _CB)])
        return 0

    lax.fori_loop(0, _NCHUNK, chunk_body, 0)


@jax.jit
def kernel(t, c, context_table, target_table):
    t_flat = t.reshape(_B)
    c_flat = c.reshape(_B * _NEG)
    k = functools.partial(
        pl.kernel,
        out_type=jax.ShapeDtypeStruct((_B, _NEG_PAD), jnp.float32),
        mesh=plsc.VectorSubcoreMesh(core_axis_name="c", subcore_axis_name="s"),
        compiler_params=pltpu.CompilerParams(needs_layout_passes=False),
        scratch_types=[
            pltpu.VMEM((_CB,), jnp.int32),
            pltpu.VMEM((_ROWS,), jnp.int32),
            pltpu.VMEM((_CB, _D), jnp.float32),
            pltpu.VMEM((_ROWS, _D), jnp.float32),
            pltpu.VMEM((_CB, _NEG_PAD), jnp.float32),
            pltpu.SemaphoreType.DMA,
        ],
    )(_body)
    out = k(t_flat, c_flat, context_table, target_table)
    return out[:, :_NEG]


# speedup vs baseline: 5.0044x; 5.0044x over previous
import functools

import jax
import jax.numpy as jnp
from jax import lax
from jax.experimental import pallas as pl
from jax.experimental.pallas import tpu as pltpu
from jax.experimental.pallas import tpu_sc as plsc

_VOCAB = 100000
_D = 128
_NEG = 50
_NEG_PAD = 64
_B = 16384

_NW = 32
_BPW = _B // _NW
_CB = 16
_NCHUNK = _BPW // _CB
_ROWS = _CB * _NEG
_GCH = [(j * 128, 128) for j in range(6)] + [(768, 32)]


_GATHER_DNUMS = lax.GatherDimensionNumbers(
    offset_dims=(), collapsed_slice_dims=(0,), start_index_map=(0,))


def _lane_rot(vec, perm):
    return lax.gather(vec, perm[:, None], _GATHER_DNUMS, slice_sizes=(1,),
                      mode=lax.GatherScatterMode.PROMISE_IN_BOUNDS)


def _body(t_ref, c_ref, ctab, ttab, out_ref,
          t_idx, c_idx, ctx_v, tgt_v, z_v, sem):
    wid = lax.axis_index("s") * 2 + lax.axis_index("c")
    lane = lax.iota(jnp.int32, 16)

    def chunk_body(ch, _):
        base = wid * _BPW + ch * _CB

        pltpu.sync_copy(t_ref.at[pl.ds(base, _CB)], t_idx)
        pltpu.sync_copy(c_ref.at[pl.ds(base * _NEG, _ROWS)], c_idx)

        copies = [pltpu.async_copy(ctab.at[t_idx], ctx_v, sem)]
        for off, sz in _GCH:
            copies.append(pltpu.async_copy(
                ttab.at[c_idx.at[pl.ds(off, sz)]],
                tgt_v.at[pl.ds(off, sz)], sem))
        for cp in copies:
            cp.wait()

        def b_body(b, _):
            ngrp = _NEG_PAD // 16
            rows = [b * _NEG + jnp.where(g * 16 + lane < _NEG,
                                         g * 16 + lane, 0)
                    for g in range(ngrp)]
            accs = [jnp.zeros((16,), jnp.float32) for _ in range(ngrp)]
            for k in range(_D // 16):
                ctx_k = ctx_v[b, pl.ds(k * 16, 16)]
                for dd in range(16):
                    perm = (lane + dd) % 16
                    bc = _lane_rot(ctx_k, perm)
                    didx = k * 16 + perm
                    for g in range(ngrp):
                        col = plsc.load_gather(tgt_v, [rows[g], didx])
                        accs[g] = accs[g] + bc * col
            zs = [jnp.where(g * 16 + lane < _NEG, accs[g],
                            jnp.float32(-jnp.inf)) for g in range(ngrp)]
            m = jnp.max(jnp.maximum(jnp.maximum(zs[0], zs[1]),
                                    jnp.maximum(zs[2], zs[3])))
            es = [jnp.exp(zj - m) for zj in zs]
            s = jnp.sum(es[0] + es[1] + es[2] + es[3])
            for j in range(ngrp):
                z_v[b, pl.ds(j * 16, 16)] = es[j] / s
            return 0

        lax.fori_loop(0, _CB, b_body, 0)
        pltpu.sync_copy(z_v, out_ref.at[pl.ds(base, _CB)])
        return 0

    lax.fori_loop(0, _NCHUNK, chunk_body, 0)


@jax.jit
def kernel(t, c, context_table, target_table):
    t_flat = t.reshape(_B)
    c_flat = c.reshape(_B * _NEG)
    k = functools.partial(
        pl.kernel,
        out_type=jax.ShapeDtypeStruct((_B, _NEG_PAD), jnp.float32),
        mesh=plsc.VectorSubcoreMesh(core_axis_name="c", subcore_axis_name="s"),
        compiler_params=pltpu.CompilerParams(needs_layout_passes=False),
        scratch_types=[
            pltpu.VMEM((_CB,), jnp.int32),
            pltpu.VMEM((_ROWS,), jnp.int32),
            pltpu.VMEM((_CB, _D), jnp.float32),
            pltpu.VMEM((_ROWS, _D), jnp.float32),
            pltpu.VMEM((_CB, _NEG_PAD), jnp.float32),
            pltpu.SemaphoreType.DMA,
        ],
    )(_body)
    out = k(t_flat, c_flat, context_table, target_table)
    return out[:, :_NEG]


# speedup vs baseline: 6.4069x; 1.2803x over previous
import functools

import jax
import jax.numpy as jnp
from jax import lax
from jax.experimental import pallas as pl
from jax.experimental.pallas import tpu as pltpu
from jax.experimental.pallas import tpu_sc as plsc

_VOCAB = 100000
_D = 128
_NEG = 50
_NEG_PAD = 64
_B = 16384

_NW = 32
_BPW = _B // _NW
_CB = 8
_NCHUNK = _BPW // _CB
_ROWS = _CB * _NEG
_GCH = [(j * 128, 128) for j in range(3)] + [(384, 16)]
_NGRP = _NEG_PAD // 16


_GATHER_DNUMS = lax.GatherDimensionNumbers(
    offset_dims=(), collapsed_slice_dims=(0,), start_index_map=(0,))


def _lane_rot(vec, perm):
    return lax.gather(vec, perm[:, None], _GATHER_DNUMS, slice_sizes=(1,),
                      mode=lax.GatherScatterMode.PROMISE_IN_BOUNDS)


def _body(t_ref, c_ref, ctab, ttab, out_ref,
          t_idx, c_idx, ctx_v, tgt_v, z_v, sems):
    wid = lax.axis_index("s") * 2 + lax.axis_index("c")
    lane = lax.iota(jnp.int32, 16)

    def fire(ch, par):
        base = wid * _BPW + ch * _CB
        pltpu.sync_copy(t_ref.at[pl.ds(base, _CB)], t_idx[par])
        pltpu.sync_copy(c_ref.at[pl.ds(base * _NEG, _ROWS)], c_idx[par])
        pltpu.async_copy(ctab.at[t_idx[par]], ctx_v[par], sems[par])
        for off, sz in _GCH:
            pltpu.async_copy(
                ttab.at[c_idx[par].at[pl.ds(off, sz)]],
                tgt_v[par].at[pl.ds(off, sz)], sems[par])

    def drain(par):
        pltpu.make_async_copy(ctab.at[t_idx[par]], ctx_v[par],
                              sems[par]).wait()
        for off, sz in _GCH:
            pltpu.make_async_copy(
                ttab.at[c_idx[par].at[pl.ds(off, sz)]],
                tgt_v[par].at[pl.ds(off, sz)], sems[par]).wait()

    def compute(ch, par):
        base = wid * _BPW + ch * _CB

        def b_body(b, _):
            rows = [b * _NEG + jnp.where(g * 16 + lane < _NEG,
                                         g * 16 + lane, 0)
                    for g in range(_NGRP)]
            accs = [jnp.zeros((16,), jnp.float32) for _ in range(_NGRP)]
            for k in range(_D // 16):
                ctx_k = ctx_v[par][b, pl.ds(k * 16, 16)]
                for dd in range(16):
                    perm = (lane + dd) % 16
                    bc = _lane_rot(ctx_k, perm)
                    didx = k * 16 + perm
                    for g in range(_NGRP):
                        col = plsc.load_gather(tgt_v[par], [rows[g], didx])
                        accs[g] = accs[g] + bc * col
            zs = [jnp.where(g * 16 + lane < _NEG, accs[g],
                            jnp.float32(-jnp.inf)) for g in range(_NGRP)]
            m = jnp.max(jnp.maximum(jnp.maximum(zs[0], zs[1]),
                                    jnp.maximum(zs[2], zs[3])))
            es = [jnp.exp(zj - m) for zj in zs]
            s = jnp.sum(es[0] + es[1] + es[2] + es[3])
            for j in range(_NGRP):
                z_v[b, pl.ds(j * 16, 16)] = es[j] / s
            return 0

        lax.fori_loop(0, _CB, b_body, 0)
        pltpu.sync_copy(z_v, out_ref.at[pl.ds(base, _CB)])

    fire(0, 0)

    def pair_body(i, _):
        ch0 = i * 2
        for par in range(2):
            ch = ch0 + par

            @pl.when(ch + 1 < _NCHUNK)
            def _():
                fire(ch + 1, 1 - par)

            drain(par)
            compute(ch, par)
        return 0

    lax.fori_loop(0, _NCHUNK // 2, pair_body, 0)


@jax.jit
def kernel(t, c, context_table, target_table):
    t_flat = t.reshape(_B)
    c_flat = c.reshape(_B * _NEG)
    k = functools.partial(
        pl.kernel,
        out_type=jax.ShapeDtypeStruct((_B, _NEG_PAD), jnp.float32),
        mesh=plsc.VectorSubcoreMesh(core_axis_name="c", subcore_axis_name="s"),
        compiler_params=pltpu.CompilerParams(needs_layout_passes=False),
        scratch_types=[
            [pltpu.VMEM((_CB,), jnp.int32) for _ in range(2)],
            [pltpu.VMEM((_ROWS,), jnp.int32) for _ in range(2)],
            [pltpu.VMEM((_CB, _D), jnp.float32) for _ in range(2)],
            [pltpu.VMEM((_ROWS, _D), jnp.float32) for _ in range(2)],
            pltpu.VMEM((_CB, _NEG_PAD), jnp.float32),
            [pltpu.SemaphoreType.DMA for _ in range(2)],
        ],
    )(_body)
    out = k(t_flat, c_flat, context_table, target_table)
    return out[:, :_NEG]


# speedup vs baseline: 7.4130x; 1.1570x over previous
import functools

import jax
import jax.numpy as jnp
from jax import lax
from jax.experimental import pallas as pl
from jax.experimental.pallas import tpu as pltpu
from jax.experimental.pallas import tpu_sc as plsc

_VOCAB = 100000
_D = 128
_NEG = 50
_NEG_PAD = 64
_ZP = 56
_B = 16384

_NW = 32
_BPW = _B // _NW
_CB = 8
_NCHUNK = _BPW // _CB
_ROWS = _CB * _NEG
_GCH = [(j * 128, 128) for j in range(3)] + [(384, 16)]
_NGRP = _NEG_PAD // 16
_SLAB = 16


_GATHER_DNUMS = lax.GatherDimensionNumbers(
    offset_dims=(), collapsed_slice_dims=(0,), start_index_map=(0,))


def _lane_rot(vec, perm):
    return lax.gather(vec, perm[:, None], _GATHER_DNUMS, slice_sizes=(1,),
                      mode=lax.GatherScatterMode.PROMISE_IN_BOUNDS)


def _body(t_ref, c_ref, ctab, ttab, out_ref,
          t_all, c_slab, ctx_v, tgt_v, z_v, sems):
    wid = lax.axis_index("s") * 2 + lax.axis_index("c")
    lane = lax.iota(jnp.int32, 16)

    pltpu.sync_copy(t_ref.at[pl.ds(wid * _BPW, _BPW)], t_all)

    def fetch_slab(ch):
        base = wid * _BPW * _NEG + (ch // _SLAB) * _SLAB * _ROWS
        pltpu.sync_copy(c_ref.at[pl.ds(base, _SLAB * _ROWS)], c_slab)

    def fire(ch, par):
        soff = (ch % _SLAB) * _ROWS
        pltpu.async_copy(ctab.at[t_all.at[pl.ds(ch * _CB, _CB)]],
                         ctx_v[par], sems[par])
        for off, sz in _GCH:
            pltpu.async_copy(
                ttab.at[c_slab.at[pl.ds(soff + off, sz)]],
                tgt_v[par].at[pl.ds(off, sz)], sems[par])

    def drain(ch, par):
        soff = (ch % _SLAB) * _ROWS
        pltpu.make_async_copy(ctab.at[t_all.at[pl.ds(ch * _CB, _CB)]],
                              ctx_v[par], sems[par]).wait()
        for off, sz in _GCH:
            pltpu.make_async_copy(
                ttab.at[c_slab.at[pl.ds(soff + off, sz)]],
                tgt_v[par].at[pl.ds(off, sz)], sems[par]).wait()

    def compute(ch, par):
        base = wid * _BPW + ch * _CB

        def b_body(b, _):
            rows = [b * _NEG + jnp.where(g * 16 + lane < _NEG,
                                         g * 16 + lane, 0)
                    for g in range(_NGRP)]
            accs = [jnp.zeros((16,), jnp.float32) for _ in range(_NGRP)]
            for k in range(_D // 16):
                ctx_k = ctx_v[par][b, pl.ds(k * 16, 16)]
                for dd in range(16):
                    perm = (lane + dd) % 16
                    bc = _lane_rot(ctx_k, perm)
                    didx = k * 16 + perm
                    for g in range(_NGRP):
                        col = plsc.load_gather(tgt_v[par], [rows[g], didx])
                        accs[g] = accs[g] + bc * col
            zs = [jnp.where(g * 16 + lane < _NEG, accs[g],
                            jnp.float32(-jnp.inf)) for g in range(_NGRP)]
            m = jnp.max(jnp.maximum(jnp.maximum(zs[0], zs[1]),
                                    jnp.maximum(zs[2], zs[3])))
            es = [jnp.exp(zj - m) for zj in zs]
            s = jnp.sum(es[0] + es[1] + es[2] + es[3])
            for j in range(_NGRP):
                z_v[pl.ds(b * _ZP + j * 16, 16)] = es[j] / s
            return 0

        lax.fori_loop(0, _CB, b_body, 0)
        pltpu.sync_copy(z_v.at[pl.ds(0, _CB * _ZP)],
                        out_ref.at[pl.ds(base * _ZP, _CB * _ZP)])

    fetch_slab(0)
    fire(0, 0)

    def pair_body(i, _):
        ch0 = i * 2
        for par in range(2):
            ch = ch0 + par
            drain(ch, par)

            @pl.when(ch + 1 < _NCHUNK)
            def _():
                @pl.when((ch + 1) % _SLAB == 0)
                def _():
                    fetch_slab(ch + 1)

                fire(ch + 1, 1 - par)

            compute(ch, par)
        return 0

    lax.fori_loop(0, _NCHUNK // 2, pair_body, 0)


@jax.jit
def kernel(t, c, context_table, target_table):
    t_flat = t.reshape(_B)
    c_flat = c.reshape(_B * _NEG)
    k = functools.partial(
        pl.kernel,
        out_type=jax.ShapeDtypeStruct((_B * _ZP,), jnp.float32),
        mesh=plsc.VectorSubcoreMesh(core_axis_name="c", subcore_axis_name="s"),
        compiler_params=pltpu.CompilerParams(needs_layout_passes=False),
        scratch_types=[
            pltpu.VMEM((_BPW,), jnp.int32),
            pltpu.VMEM((_SLAB * _ROWS,), jnp.int32),
            [pltpu.VMEM((_CB, _D), jnp.float32) for _ in range(2)],
            [pltpu.VMEM((_ROWS, _D), jnp.float32) for _ in range(2)],
            pltpu.VMEM((_CB * _ZP + 8,), jnp.float32),
            [pltpu.SemaphoreType.DMA for _ in range(2)],
        ],
    )(_body)
    out = k(t_flat, c_flat, context_table, target_table)
    return out.reshape(_B, _ZP)[:, :_NEG]
